# Initial kernel scaffold; baseline (speedup 1.0000x reference)
import functools

import jax
import jax.numpy as jnp
from jax import lax
from jax.experimental import pallas as pl
from jax.experimental.pallas import tpu as pltpu
from jax.experimental.pallas import tpu_sc as plsc

B, N, D_IN, D, H, K, PH = 4, 512, 3, 64, 4, 16, 64
PPAD = 16
R = B * N * K
_HI = jax.lax.Precision.HIGHEST


def _stage_a_body(xp_ref, posp_ref, wqkv_ref, q_ref, kv_ref, idx_ref):
    b = pl.program_id(0)
    xp = xp_ref[0]
    P = posp_ref[0]
    qkv = jnp.dot(xp, wqkv_ref[...],
                  preferred_element_type=jnp.float32, precision=_HI)
    q_ref[0] = qkv[:, :D]
    kv_ref[0, :, :D] = qkv[:, D:2 * D]
    kv_ref[0, :, D:] = qkv[:, 2 * D:]

    G = lax.dot_general(P, P, (((1,), (1,)), ((), ())),
                        preferred_element_type=jnp.float32, precision=_HI)
    r = jnp.sum(P * P, axis=1, keepdims=True)
    ii = lax.broadcasted_iota(jnp.float32, (N, N), 0)
    jj = lax.broadcasted_iota(jnp.float32, (N, N), 1)
    eye = (ii == jj).astype(jnp.float32)
    rrow = jnp.sum(G * eye, axis=0, keepdims=True)
    d2 = r + rrow - 2.0 * G

    off = jnp.int32(N) * b
    for t in range(K):
        m = jnp.min(d2, axis=1, keepdims=True)
        cand = jnp.where(d2 <= m, jj, jnp.float32(N))
        amin = jnp.min(cand, axis=1, keepdims=True)
        idx_ref[0, :, pl.ds(t, 1)] = amin.astype(jnp.int32) + off
        d2 = jnp.where(jj == amin, jnp.float32(1e30), d2)


def _stage_a(xp, posp, wqkv_p):
    return pl.pallas_call(
        _stage_a_body,
        grid=(B,),
        in_specs=[
            pl.BlockSpec((1, N, PPAD), lambda b: (b, 0, 0)),
            pl.BlockSpec((1, N, PPAD), lambda b: (b, 0, 0)),
            pl.BlockSpec((PPAD, 3 * D), lambda b: (0, 0)),
        ],
        out_specs=[
            pl.BlockSpec((1, N, D), lambda b: (b, 0, 0)),
            pl.BlockSpec((1, N, 2 * D), lambda b: (b, 0, 0)),
            pl.BlockSpec((1, N, K), lambda b: (b, 0, 0)),
        ],
        out_shape=[
            jax.ShapeDtypeStruct((B, N, D), jnp.float32),
            jax.ShapeDtypeStruct((B, N, 2 * D), jnp.float32),
            jax.ShapeDtypeStruct((B, N, K), jnp.int32),
        ],
    )(xp, posp, wqkv_p)


_info = plsc.get_sparse_core_info()
_NC, _NS = _info.num_cores, _info.num_subcores
_NW = _NC * _NS
_RPW = R // _NW
_CH = 128
_NCHUNK = _RPW // _CH


@functools.partial(
    pl.kernel,
    out_type=[
        jax.ShapeDtypeStruct((R, 2 * D), jnp.float32),
        jax.ShapeDtypeStruct((R, PPAD), jnp.float32),
    ],
    mesh=plsc.VectorSubcoreMesh(core_axis_name="c", subcore_axis_name="s"),
    scratch_types=[
        pltpu.VMEM((_CH,), jnp.int32),
        pltpu.VMEM((_CH, 2 * D), jnp.float32),
        pltpu.VMEM((_CH, PPAD), jnp.float32),
        pltpu.SemaphoreType.DMA,
    ],
)
def _stage_b(kv_hbm, pos_hbm, idx_hbm, kvout_hbm, posout_hbm,
             idx_v, kvrow_v, posrow_v, sem):
    wid = lax.axis_index("s") * _NC + lax.axis_index("c")
    base0 = wid * _RPW
    for c in range(_NCHUNK):
        base = base0 + c * _CH
        pltpu.sync_copy(idx_hbm.at[pl.ds(base, _CH)], idx_v)
        pltpu.async_copy(kv_hbm.at[idx_v], kvrow_v, sem).wait()
        pltpu.async_copy(pos_hbm.at[idx_v], posrow_v, sem).wait()
        pltpu.sync_copy(kvrow_v, kvout_hbm.at[pl.ds(base, _CH)])
        pltpu.sync_copy(posrow_v, posout_hbm.at[pl.ds(base, _CH)])


_QB = 256


def _stage_c_body(q_ref, pq_ref, kvs_ref, pss_ref, wp1_ref, bp1_ref,
                  wp2_ref, bp2_ref, wa1_ref, ba1_ref, wa2_ref, ba2_ref,
                  out_ref):
    q = q_ref[...]
    pq = pq_ref[...]
    wp1 = wp1_ref[...]
    wp2 = wp2_ref[...]
    wa1 = wa1_ref[...]
    wa2 = wa2_ref[...]
    bp1 = bp1_ref[...]
    bp2 = bp2_ref[...]
    ba1 = ba1_ref[...]
    ba2 = ba2_ref[...]

    m = num = den = None
    for k in range(K):
        g = kvs_ref[k]
        pj = pss_ref[k]
        rel = pq - pj
        pe = jnp.dot(jnp.maximum(
            jnp.dot(rel, wp1, preferred_element_type=jnp.float32,
                    precision=_HI) + bp1, 0.0),
            wp2, preferred_element_type=jnp.float32, precision=_HI) + bp2
        qk = q - g[:, :D]
        h = jnp.maximum(
            jnp.dot(qk + pe, wa1, preferred_element_type=jnp.float32,
                    precision=_HI) + ba1, 0.0)
        sim = jnp.dot(h, wa2, preferred_element_type=jnp.float32,
                      precision=_HI) + ba2
        vv = g[:, D:] + pe
        if k == 0:
            m = sim
            num = vv
            den = jnp.ones_like(sim)
        else:
            m2 = jnp.maximum(m, sim)
            a = jnp.exp(m - m2)
            e = jnp.exp(sim - m2)
            num = num * a + e * vv
            den = den * a + e
            m = m2
    out_ref[...] = num / den


def _stage_c(q_flat, pos_flat, kv_sel, pos_sel, wp1_p, bp1, wp2, bp2,
             wa1, ba1, wa2, ba2):
    grid = (B * N) // _QB
    return pl.pallas_call(
        _stage_c_body,
        grid=(grid,),
        in_specs=[
            pl.BlockSpec((_QB, D), lambda i: (i, 0)),
            pl.BlockSpec((_QB, PPAD), lambda i: (i, 0)),
            pl.BlockSpec((K, _QB, 2 * D), lambda i: (0, i, 0)),
            pl.BlockSpec((K, _QB, PPAD), lambda i: (0, i, 0)),
            pl.BlockSpec((PPAD, PH), lambda i: (0, 0)),
            pl.BlockSpec((1, PH), lambda i: (0, 0)),
            pl.BlockSpec((PH, D), lambda i: (0, 0)),
            pl.BlockSpec((1, D), lambda i: (0, 0)),
            pl.BlockSpec((D, D * H), lambda i: (0, 0)),
            pl.BlockSpec((1, D * H), lambda i: (0, 0)),
            pl.BlockSpec((D * H, D), lambda i: (0, 0)),
            pl.BlockSpec((1, D), lambda i: (0, 0)),
        ],
        out_specs=pl.BlockSpec((_QB, D), lambda i: (i, 0)),
        out_shape=jax.ShapeDtypeStruct((B * N, D), jnp.float32),
    )(q_flat, pos_flat, kv_sel, pos_sel, wp1_p, bp1, wp2, bp2,
      wa1, ba1, wa2, ba2)


def kernel(x, pos, Wqkv, Wp1, bp1, Wp2, bp2, Wa1, ba1, Wa2, ba2):
    xp = jnp.pad(x, ((0, 0), (0, 0), (0, PPAD - D_IN)))
    posp = jnp.pad(pos, ((0, 0), (0, 0), (0, PPAD - D_IN)))
    wqkv_p = jnp.pad(Wqkv, ((0, PPAD - D_IN), (0, 0)))
    wp1_p = jnp.pad(Wp1, ((0, PPAD - D_IN), (0, 0)))

    q, kv, idx = _stage_a(xp, posp, wqkv_p)

    idx_t = jnp.transpose(idx, (2, 0, 1)).reshape(R)
    kv_flat = kv.reshape(B * N, 2 * D)
    pos_flat = posp.reshape(B * N, PPAD)
    kv_sel, pos_sel = _stage_b(kv_flat, pos_flat, idx_t)

    agg = _stage_c(
        q.reshape(B * N, D), pos_flat,
        kv_sel.reshape(K, B * N, 2 * D), pos_sel.reshape(K, B * N, PPAD),
        wp1_p, bp1.reshape(1, PH), Wp2, bp2.reshape(1, D),
        Wa1, ba1.reshape(1, D * H), Wa2, ba2.reshape(1, D))
    return agg.reshape(B, N, D)


# speedup vs baseline: 3.2338x; 3.2338x over previous
import functools

import jax
import jax.numpy as jnp
from jax import lax
from jax.experimental import pallas as pl
from jax.experimental.pallas import tpu as pltpu
from jax.experimental.pallas import tpu_sc as plsc

B, N, D_IN, D, H, K, PH = 4, 512, 3, 64, 4, 16, 64
PPAD = 16
R = B * N * K
_HI = jax.lax.Precision.HIGHEST


def _stage_a_body(xp_ref, posp_ref, wqkv_ref, q_ref, kv_ref, idx_ref,
                  rel_ref):
    b = pl.program_id(0)
    xp = xp_ref[0]
    P = posp_ref[0]
    qkv = jnp.dot(xp, wqkv_ref[...],
                  preferred_element_type=jnp.float32, precision=_HI)
    q_ref[0] = qkv[:, :D]
    kv_ref[0, :, :D] = qkv[:, D:2 * D]
    kv_ref[0, :, D:] = qkv[:, 2 * D:]

    G = lax.dot_general(P, P, (((1,), (1,)), ((), ())),
                        preferred_element_type=jnp.float32, precision=_HI)
    r = jnp.sum(P * P, axis=1, keepdims=True)
    ii = lax.broadcasted_iota(jnp.int32, (N, N), 0)
    jji = lax.broadcasted_iota(jnp.int32, (N, N), 1)
    jj = jji.astype(jnp.float32)
    eye = (ii == jji).astype(jnp.float32)
    rrow = jnp.sum(G * eye, axis=0, keepdims=True)
    d2 = r + rrow - 2.0 * G

    off = jnp.int32(N) * b
    for t in range(K):
        m = jnp.min(d2, axis=1, keepdims=True)
        cand = jnp.where(d2 <= m, jj, jnp.float32(N))
        amin = jnp.min(cand, axis=1, keepdims=True)
        idx_ref[0, :, pl.ds(t, 1)] = amin.astype(jnp.int32) + off
        onehot = (jj == amin).astype(jnp.float32)
        posj = lax.dot_general(onehot, P, (((1,), (0,)), ((), ())),
                               preferred_element_type=jnp.float32,
                               precision=_HI)
        rel_ref[t, 0] = P - posj
        d2 = jnp.where(jj == amin, jnp.float32(1e30), d2)


def _stage_a(xp, posp, wqkv_p):
    return pl.pallas_call(
        _stage_a_body,
        grid=(B,),
        in_specs=[
            pl.BlockSpec((1, N, PPAD), lambda b: (b, 0, 0)),
            pl.BlockSpec((1, N, PPAD), lambda b: (b, 0, 0)),
            pl.BlockSpec((PPAD, 3 * D), lambda b: (0, 0)),
        ],
        out_specs=[
            pl.BlockSpec((1, N, D), lambda b: (b, 0, 0)),
            pl.BlockSpec((1, N, 2 * D), lambda b: (b, 0, 0)),
            pl.BlockSpec((1, N, K), lambda b: (b, 0, 0)),
            pl.BlockSpec((K, 1, N, PPAD), lambda b: (0, b, 0, 0)),
        ],
        out_shape=[
            jax.ShapeDtypeStruct((B, N, D), jnp.float32),
            jax.ShapeDtypeStruct((B, N, 2 * D), jnp.float32),
            jax.ShapeDtypeStruct((B, N, K), jnp.int32),
            jax.ShapeDtypeStruct((K, B, N, PPAD), jnp.float32),
        ],
    )(xp, posp, wqkv_p)


_NC, _NS = 2, 16
_NW = _NC * _NS
_RPW = R // _NW
_CH = 128
_NCHUNK = _RPW // _CH


@functools.cache
def _make_stage_b():
    mesh = plsc.VectorSubcoreMesh(core_axis_name="c", subcore_axis_name="s",
                                  num_cores=_NC, num_subcores=_NS)

    @functools.partial(
        pl.kernel,
        out_type=jax.ShapeDtypeStruct((R, 2 * D), jnp.float32),
        mesh=mesh,
        scratch_types=[
            pltpu.VMEM((_CH,), jnp.int32),
            pltpu.VMEM((_CH, 2 * D), jnp.float32),
            pltpu.SemaphoreType.DMA,
        ],
    )
    def _stage_b(kv_hbm, idx_hbm, kvout_hbm, idx_v, kvrow_v, sem):
        wid = lax.axis_index("s") * _NC + lax.axis_index("c")
        base0 = wid * _RPW
        for c in range(_NCHUNK):
            base = base0 + c * _CH
            pltpu.sync_copy(idx_hbm.at[pl.ds(base, _CH)], idx_v)
            pltpu.async_copy(kv_hbm.at[idx_v], kvrow_v, sem).wait()
            pltpu.sync_copy(kvrow_v, kvout_hbm.at[pl.ds(base, _CH)])

    return _stage_b


_QB = 256


def _stage_c_body(q_ref, kvs_ref, rel_ref, wp1_ref, bp1_ref,
                  wp2_ref, bp2_ref, wa1_ref, ba1_ref, wa2_ref, ba2_ref,
                  out_ref):
    q = q_ref[...]
    wp1 = wp1_ref[...]
    wp2 = wp2_ref[...]
    wa1 = wa1_ref[...]
    wa2 = wa2_ref[...]
    bp1 = bp1_ref[...]
    bp2 = bp2_ref[...]
    ba1 = ba1_ref[...]
    ba2 = ba2_ref[...]

    m = num = den = None
    for k in range(K):
        g = kvs_ref[k]
        rel = rel_ref[k]
        pe = jnp.dot(jnp.maximum(
            jnp.dot(rel, wp1, preferred_element_type=jnp.float32,
                    precision=_HI) + bp1, 0.0),
            wp2, preferred_element_type=jnp.float32, precision=_HI) + bp2
        qk = q - g[:, :D]
        h = jnp.maximum(
            jnp.dot(qk + pe, wa1, preferred_element_type=jnp.float32,
                    precision=_HI) + ba1, 0.0)
        sim = jnp.dot(h, wa2, preferred_element_type=jnp.float32,
                      precision=_HI) + ba2
        vv = g[:, D:] + pe
        if k == 0:
            m = sim
            num = vv
            den = jnp.ones_like(sim)
        else:
            m2 = jnp.maximum(m, sim)
            a = jnp.exp(m - m2)
            e = jnp.exp(sim - m2)
            num = num * a + e * vv
            den = den * a + e
            m = m2
    out_ref[...] = num / den


def _stage_c(q_flat, kv_sel, rel_sel, wp1_p, bp1, wp2, bp2,
             wa1, ba1, wa2, ba2):
    grid = (B * N) // _QB
    return pl.pallas_call(
        _stage_c_body,
        grid=(grid,),
        in_specs=[
            pl.BlockSpec((_QB, D), lambda i: (i, 0)),
            pl.BlockSpec((K, _QB, 2 * D), lambda i: (0, i, 0)),
            pl.BlockSpec((K, _QB, PPAD), lambda i: (0, i, 0)),
            pl.BlockSpec((PPAD, PH), lambda i: (0, 0)),
            pl.BlockSpec((1, PH), lambda i: (0, 0)),
            pl.BlockSpec((PH, D), lambda i: (0, 0)),
            pl.BlockSpec((1, D), lambda i: (0, 0)),
            pl.BlockSpec((D, D * H), lambda i: (0, 0)),
            pl.BlockSpec((1, D * H), lambda i: (0, 0)),
            pl.BlockSpec((D * H, D), lambda i: (0, 0)),
            pl.BlockSpec((1, D), lambda i: (0, 0)),
        ],
        out_specs=pl.BlockSpec((_QB, D), lambda i: (i, 0)),
        out_shape=jax.ShapeDtypeStruct((B * N, D), jnp.float32),
    )(q_flat, kv_sel, rel_sel, wp1_p, bp1, wp2, bp2,
      wa1, ba1, wa2, ba2)


def kernel(x, pos, Wqkv, Wp1, bp1, Wp2, bp2, Wa1, ba1, Wa2, ba2):
    xp = jnp.pad(x, ((0, 0), (0, 0), (0, PPAD - D_IN)))
    posp = jnp.pad(pos, ((0, 0), (0, 0), (0, PPAD - D_IN)))
    wqkv_p = jnp.pad(Wqkv, ((0, PPAD - D_IN), (0, 0)))
    wp1_p = jnp.pad(Wp1, ((0, PPAD - D_IN), (0, 0)))

    q, kv, idx, rel = _stage_a(xp, posp, wqkv_p)

    idx_t = jnp.transpose(idx, (2, 0, 1)).reshape(R)
    kv_flat = kv.reshape(B * N, 2 * D)
    kv_sel = _make_stage_b()(kv_flat, idx_t)

    agg = _stage_c(
        q.reshape(B * N, D),
        kv_sel.reshape(K, B * N, 2 * D), rel.reshape(K, B * N, PPAD),
        wp1_p, bp1.reshape(1, PH), Wp2, bp2.reshape(1, D),
        Wa1, ba1.reshape(1, D * H), Wa2, ba2.reshape(1, D))
    return agg.reshape(B, N, D)


# speedup vs baseline: 4.5736x; 1.4143x over previous
import functools

import jax
import jax.numpy as jnp
from jax import lax
from jax.experimental import pallas as pl
from jax.experimental.pallas import tpu as pltpu
from jax.experimental.pallas import tpu_sc as plsc

B, N, D_IN, D, H, K, PH = 4, 512, 3, 64, 4, 16, 64
PPAD = 16
TW = 4 * D
R = B * N * K
_HI = jax.lax.Precision.HIGHEST


def _stage_a_body(xp_ref, posp_ref, post_ref, wqkv_ref, wp1_ref,
                  tab_ref, idx_ref):
    b = pl.program_id(0)
    xp = xp_ref[0]
    P = posp_ref[0]
    PT = post_ref[0]
    qkv = jnp.dot(xp, wqkv_ref[...],
                  preferred_element_type=jnp.float32, precision=_HI)
    tab_ref[0, :, :3 * D] = qkv
    tab_ref[0, :, 3 * D:] = jnp.dot(
        P, wp1_ref[...], preferred_element_type=jnp.float32, precision=_HI)

    t0 = P[:, 0:1] - PT[0:1, :]
    t1 = P[:, 1:2] - PT[1:2, :]
    t2 = P[:, 2:3] - PT[2:3, :]
    nd = jnp.sqrt(t0 * t0 + t1 * t1 + t2 * t2)

    jj = lax.broadcasted_iota(jnp.int32, (N, N), 1).astype(jnp.float32)

    off = jnp.int32(N) * b
    for t in range(K):
        m = jnp.min(nd, axis=1, keepdims=True)
        cand = jnp.where(nd <= m, jj, jnp.float32(N))
        amin = jnp.min(cand, axis=1, keepdims=True)
        idx_ref[0, :, pl.ds(t, 1)] = amin.astype(jnp.int32) + off
        nd = jnp.where(jj == amin, jnp.float32(3e38), nd)


def _stage_a(xp, posp, post, wqkv_p, wp1_p):
    return pl.pallas_call(
        _stage_a_body,
        grid=(B,),
        in_specs=[
            pl.BlockSpec((1, N, PPAD), lambda b: (b, 0, 0)),
            pl.BlockSpec((1, N, PPAD), lambda b: (b, 0, 0)),
            pl.BlockSpec((1, PPAD, N), lambda b: (b, 0, 0)),
            pl.BlockSpec((PPAD, 3 * D), lambda b: (0, 0)),
            pl.BlockSpec((PPAD, PH), lambda b: (0, 0)),
        ],
        out_specs=[
            pl.BlockSpec((1, N, TW), lambda b: (b, 0, 0)),
            pl.BlockSpec((1, N, K), lambda b: (b, 0, 0)),
        ],
        out_shape=[
            jax.ShapeDtypeStruct((B, N, TW), jnp.float32),
            jax.ShapeDtypeStruct((B, N, K), jnp.int32),
        ],
    )(xp, posp, post, wqkv_p, wp1_p)


_NC, _NS = 2, 16
_NW = _NC * _NS
_RPW = R // _NW
_CH = 128
_NCHUNK = _RPW // _CH


@functools.cache
def _make_stage_b():
    mesh = plsc.VectorSubcoreMesh(core_axis_name="c", subcore_axis_name="s",
                                  num_cores=_NC, num_subcores=_NS)

    @functools.partial(
        pl.kernel,
        out_type=jax.ShapeDtypeStruct((R, TW), jnp.float32),
        mesh=mesh,
        scratch_types=[
            pltpu.VMEM((_CH,), jnp.int32),
            pltpu.VMEM((_CH, TW), jnp.float32),
            pltpu.SemaphoreType.DMA,
        ],
    )
    def _stage_b(tab_hbm, idx_hbm, out_hbm, idx_v, row_v, sem):
        wid = lax.axis_index("s") * _NC + lax.axis_index("c")
        base0 = wid * _RPW
        for c in range(_NCHUNK):
            base = base0 + c * _CH
            pltpu.sync_copy(idx_hbm.at[pl.ds(base, _CH)], idx_v)
            pltpu.async_copy(tab_hbm.at[idx_v], row_v, sem).wait()
            pltpu.sync_copy(row_v, out_hbm.at[pl.ds(base, _CH)])

    return _stage_b


_QB = 256


def _stage_c_body(tq_ref, gath_ref, wp2_ref, bp2_ref, wa1_ref, ba1_ref,
                  wa2_ref, ba2_ref, bp1_ref, out_ref):
    tq = tq_ref[...]
    q = tq[:, :D]
    a1q = tq[:, 3 * D:]
    gf = gath_ref[...].reshape(K * _QB, TW)
    kk = gf[:, D:2 * D]
    vv0 = gf[:, 2 * D:3 * D]
    a1j = gf[:, 3 * D:]

    qb = jnp.broadcast_to(q[None], (K, _QB, D)).reshape(K * _QB, D)
    a1qb = jnp.broadcast_to(a1q[None], (K, _QB, D)).reshape(K * _QB, D)

    pre = jnp.maximum(a1qb - a1j + bp1_ref[...], 0.0)
    pe = jnp.dot(pre, wp2_ref[...],
                 preferred_element_type=jnp.float32, precision=_HI) \
        + bp2_ref[...]
    h = jnp.maximum(
        jnp.dot(qb - kk + pe, wa1_ref[...],
                preferred_element_type=jnp.float32, precision=_HI)
        + ba1_ref[...], 0.0)
    sim = jnp.dot(h, wa2_ref[...],
                  preferred_element_type=jnp.float32, precision=_HI) \
        + ba2_ref[...]
    vv = vv0 + pe

    sim3 = sim.reshape(K, _QB, D)
    vv3 = vv.reshape(K, _QB, D)
    m = sim3[0]
    num = vv3[0]
    den = jnp.ones_like(m)
    for k in range(1, K):
        s = sim3[k]
        m2 = jnp.maximum(m, s)
        a = jnp.exp(m - m2)
        e = jnp.exp(s - m2)
        num = num * a + e * vv3[k]
        den = den * a + e
        m = m2
    out_ref[...] = num / den


def _stage_c(tab_flat, gath, wp2, bp2, wa1, ba1, wa2, ba2, bp1):
    grid = (B * N) // _QB
    return pl.pallas_call(
        _stage_c_body,
        grid=(grid,),
        in_specs=[
            pl.BlockSpec((_QB, TW), lambda i: (i, 0)),
            pl.BlockSpec((K, _QB, TW), lambda i: (0, i, 0)),
            pl.BlockSpec((PH, D), lambda i: (0, 0)),
            pl.BlockSpec((1, D), lambda i: (0, 0)),
            pl.BlockSpec((D, D * H), lambda i: (0, 0)),
            pl.BlockSpec((1, D * H), lambda i: (0, 0)),
            pl.BlockSpec((D * H, D), lambda i: (0, 0)),
            pl.BlockSpec((1, D), lambda i: (0, 0)),
            pl.BlockSpec((1, PH), lambda i: (0, 0)),
        ],
        out_specs=pl.BlockSpec((_QB, D), lambda i: (i, 0)),
        out_shape=jax.ShapeDtypeStruct((B * N, D), jnp.float32),
    )(tab_flat, gath, wp2, bp2, wa1, ba1, wa2, ba2, bp1)


def kernel(x, pos, Wqkv, Wp1, bp1, Wp2, bp2, Wa1, ba1, Wa2, ba2):
    xp = jnp.pad(x, ((0, 0), (0, 0), (0, PPAD - D_IN)))
    posp = jnp.pad(pos, ((0, 0), (0, 0), (0, PPAD - D_IN)))
    post = jnp.transpose(posp, (0, 2, 1))
    wqkv_p = jnp.pad(Wqkv, ((0, PPAD - D_IN), (0, 0)))
    wp1_p = jnp.pad(Wp1, ((0, PPAD - D_IN), (0, 0)))

    tab, idx = _stage_a(xp, posp, post, wqkv_p, wp1_p)

    idx_t = jnp.transpose(idx, (2, 0, 1)).reshape(R)
    tab_flat = tab.reshape(B * N, TW)
    gath = _make_stage_b()(tab_flat, idx_t)

    agg = _stage_c(
        tab_flat, gath.reshape(K, B * N, TW),
        Wp2, bp2.reshape(1, D), Wa1, ba1.reshape(1, D * H),
        Wa2, ba2.reshape(1, D), bp1.reshape(1, PH))
    return agg.reshape(B, N, D)


# speedup vs baseline: 8.5231x; 1.8635x over previous
import functools

import jax
import jax.numpy as jnp
from jax import lax
from jax.experimental import pallas as pl
from jax.experimental.pallas import tpu as pltpu
from jax.experimental.pallas import tpu_sc as plsc

B, N, D_IN, D, H, K, PH = 4, 512, 3, 64, 4, 16, 64
PPAD = 16
TW = 4 * D
R = B * N * K
_HI = jax.lax.Precision.HIGHEST
_MLP = jax.lax.Precision.DEFAULT


def _stage_a_body(xp_ref, posp_ref, post_ref, wqkv_ref, wp1_ref,
                  tab_ref, idx_ref):
    b = pl.program_id(0)
    xp = xp_ref[0]
    P = posp_ref[0]
    PT = post_ref[0]
    qkv = jnp.dot(xp, wqkv_ref[...],
                  preferred_element_type=jnp.float32, precision=_HI)
    tab_ref[0, :, :3 * D] = qkv
    tab_ref[0, :, 3 * D:] = jnp.dot(
        P, wp1_ref[...], preferred_element_type=jnp.float32, precision=_HI)

    t0 = P[:, 0:1] - PT[0:1, :]
    t1 = P[:, 1:2] - PT[1:2, :]
    t2 = P[:, 2:3] - PT[2:3, :]
    nd = jnp.sqrt(t0 * t0 + t1 * t1 + t2 * t2)

    jj = lax.broadcasted_iota(jnp.int32, (N, N), 1).astype(jnp.float32)

    off = jnp.int32(N) * b
    for t in range(K):
        m = jnp.min(nd, axis=1, keepdims=True)
        cand = jnp.where(nd <= m, jj, jnp.float32(N))
        amin = jnp.min(cand, axis=1, keepdims=True)
        idx_ref[0, :, pl.ds(t, 1)] = amin.astype(jnp.int32) + off
        nd = jnp.where(jj == amin, jnp.float32(3e38), nd)


def _stage_a(xp, posp, post, wqkv_p, wp1_p):
    return pl.pallas_call(
        _stage_a_body,
        grid=(B,),
        in_specs=[
            pl.BlockSpec((1, N, PPAD), lambda b: (b, 0, 0)),
            pl.BlockSpec((1, N, PPAD), lambda b: (b, 0, 0)),
            pl.BlockSpec((1, PPAD, N), lambda b: (b, 0, 0)),
            pl.BlockSpec((PPAD, 3 * D), lambda b: (0, 0)),
            pl.BlockSpec((PPAD, PH), lambda b: (0, 0)),
        ],
        out_specs=[
            pl.BlockSpec((1, N, TW), lambda b: (b, 0, 0)),
            pl.BlockSpec((1, N, K), lambda b: (b, 0, 0)),
        ],
        out_shape=[
            jax.ShapeDtypeStruct((B, N, TW), jnp.float32),
            jax.ShapeDtypeStruct((B, N, K), jnp.int32),
        ],
    )(xp, posp, post, wqkv_p, wp1_p)


_NC, _NS = 2, 16
_NW = _NC * _NS
_RPW = R // _NW
_CH = 128
_NCHUNK = _RPW // _CH


@functools.cache
def _make_stage_b():
    mesh = plsc.VectorSubcoreMesh(core_axis_name="c", subcore_axis_name="s",
                                  num_cores=_NC, num_subcores=_NS)

    @functools.partial(
        pl.kernel,
        out_type=jax.ShapeDtypeStruct((R, TW), jnp.float32),
        mesh=mesh,
        scratch_types=[
            pltpu.VMEM((_CH,), jnp.int32),
            pltpu.VMEM((_CH, TW), jnp.float32),
            pltpu.SemaphoreType.DMA,
        ],
    )
    def _stage_b(tab_hbm, idx_hbm, out_hbm, idx_v, row_v, sem):
        wid = lax.axis_index("s") * _NC + lax.axis_index("c")
        base0 = wid * _RPW
        for c in range(_NCHUNK):
            base = base0 + c * _CH
            pltpu.sync_copy(idx_hbm.at[pl.ds(base, _CH)], idx_v)
            pltpu.async_copy(tab_hbm.at[idx_v], row_v, sem).wait()
            pltpu.sync_copy(row_v, out_hbm.at[pl.ds(base, _CH)])

    return _stage_b


_QB = 256


def _stage_c_body(tq_ref, gath_ref, wp2_ref, bp2_ref, wa1_ref, ba1_ref,
                  wa2_ref, ba2_ref, bp1_ref, out_ref):
    tq = tq_ref[...]
    q = tq[:, :D]
    a1q = tq[:, 3 * D:]
    gf = gath_ref[...].reshape(K * _QB, TW)
    kk = gf[:, D:2 * D]
    vv0 = gf[:, 2 * D:3 * D]
    a1j = gf[:, 3 * D:]

    qb = jnp.broadcast_to(q[None], (K, _QB, D)).reshape(K * _QB, D)
    a1qb = jnp.broadcast_to(a1q[None], (K, _QB, D)).reshape(K * _QB, D)

    pre = jnp.maximum(a1qb - a1j + bp1_ref[...], 0.0)
    pe = jnp.dot(pre, wp2_ref[...],
                 preferred_element_type=jnp.float32, precision=_MLP) \
        + bp2_ref[...]
    h = jnp.maximum(
        jnp.dot(qb - kk + pe, wa1_ref[...],
                preferred_element_type=jnp.float32, precision=_MLP)
        + ba1_ref[...], 0.0)
    sim = jnp.dot(h, wa2_ref[...],
                  preferred_element_type=jnp.float32, precision=_MLP) \
        + ba2_ref[...]
    vv = vv0 + pe

    sim3 = sim.reshape(K, _QB, D)
    vv3 = vv.reshape(K, _QB, D)
    m = sim3[0]
    num = vv3[0]
    den = jnp.ones_like(m)
    for k in range(1, K):
        s = sim3[k]
        m2 = jnp.maximum(m, s)
        a = jnp.exp(m - m2)
        e = jnp.exp(s - m2)
        num = num * a + e * vv3[k]
        den = den * a + e
        m = m2
    out_ref[...] = num / den


def _stage_c(tab_flat, gath, wp2, bp2, wa1, ba1, wa2, ba2, bp1):
    grid = (B * N) // _QB
    return pl.pallas_call(
        _stage_c_body,
        grid=(grid,),
        in_specs=[
            pl.BlockSpec((_QB, TW), lambda i: (i, 0)),
            pl.BlockSpec((K, _QB, TW), lambda i: (0, i, 0)),
            pl.BlockSpec((PH, D), lambda i: (0, 0)),
            pl.BlockSpec((1, D), lambda i: (0, 0)),
            pl.BlockSpec((D, D * H), lambda i: (0, 0)),
            pl.BlockSpec((1, D * H), lambda i: (0, 0)),
            pl.BlockSpec((D * H, D), lambda i: (0, 0)),
            pl.BlockSpec((1, D), lambda i: (0, 0)),
            pl.BlockSpec((1, PH), lambda i: (0, 0)),
        ],
        out_specs=pl.BlockSpec((_QB, D), lambda i: (i, 0)),
        out_shape=jax.ShapeDtypeStruct((B * N, D), jnp.float32),
    )(tab_flat, gath, wp2, bp2, wa1, ba1, wa2, ba2, bp1)


def kernel(x, pos, Wqkv, Wp1, bp1, Wp2, bp2, Wa1, ba1, Wa2, ba2):
    xp = jnp.pad(x, ((0, 0), (0, 0), (0, PPAD - D_IN)))
    posp = jnp.pad(pos, ((0, 0), (0, 0), (0, PPAD - D_IN)))
    post = jnp.transpose(posp, (0, 2, 1))
    wqkv_p = jnp.pad(Wqkv, ((0, PPAD - D_IN), (0, 0)))
    wp1_p = jnp.pad(Wp1, ((0, PPAD - D_IN), (0, 0)))

    tab, idx = _stage_a(xp, posp, post, wqkv_p, wp1_p)

    idx_t = jnp.transpose(idx, (2, 0, 1)).reshape(R)
    tab_flat = tab.reshape(B * N, TW)
    gath = _make_stage_b()(tab_flat, idx_t)

    agg = _stage_c(
        tab_flat, gath.reshape(K, B * N, TW),
        Wp2, bp2.reshape(1, D), Wa1, ba1.reshape(1, D * H),
        Wa2, ba2.reshape(1, D), bp1.reshape(1, PH))
    return agg.reshape(B, N, D)


# speedup vs baseline: 8.9582x; 1.0510x over previous
import functools

import jax
import jax.numpy as jnp
from jax import lax
from jax.experimental import pallas as pl
from jax.experimental.pallas import tpu as pltpu
from jax.experimental.pallas import tpu_sc as plsc

B, N, D_IN, D, H, K, PH = 4, 512, 3, 64, 4, 16, 64
PPAD = 16
TW = 4 * D
R = B * N * K
_HI = jax.lax.Precision.HIGHEST
_MLP = jax.lax.Precision.DEFAULT


def _stage_a_body(xp_ref, posp_ref, post_ref, wqkv_ref, wp1_ref,
                  tab_ref, idx_ref):
    b = pl.program_id(0)
    xp = xp_ref[0]
    P = posp_ref[0]
    PT = post_ref[0]
    qkv = jnp.dot(xp, wqkv_ref[...],
                  preferred_element_type=jnp.float32, precision=_HI)
    tab_ref[0, :, :3 * D] = qkv
    tab_ref[0, :, 3 * D:] = jnp.dot(
        P, wp1_ref[...], preferred_element_type=jnp.float32, precision=_HI)

    t0 = P[:, 0:1] - PT[0:1, :]
    t1 = P[:, 1:2] - PT[1:2, :]
    t2 = P[:, 2:3] - PT[2:3, :]
    nd = jnp.sqrt(t0 * t0 + t1 * t1 + t2 * t2)

    jj = lax.broadcasted_iota(jnp.int32, (N, N), 1).astype(jnp.float32)

    off = jnp.int32(N) * b
    for t in range(K):
        m = jnp.min(nd, axis=1, keepdims=True)
        cand = jnp.where(nd <= m, jj, jnp.float32(N))
        amin = jnp.min(cand, axis=1, keepdims=True)
        idx_ref[0, :, pl.ds(t, 1)] = amin.astype(jnp.int32) + off
        nd = jnp.where(jj == amin, jnp.float32(3e38), nd)


def _stage_a(xp, posp, post, wqkv_p, wp1_p):
    return pl.pallas_call(
        _stage_a_body,
        grid=(B,),
        in_specs=[
            pl.BlockSpec((1, N, PPAD), lambda b: (b, 0, 0)),
            pl.BlockSpec((1, N, PPAD), lambda b: (b, 0, 0)),
            pl.BlockSpec((1, PPAD, N), lambda b: (b, 0, 0)),
            pl.BlockSpec((PPAD, 3 * D), lambda b: (0, 0)),
            pl.BlockSpec((PPAD, PH), lambda b: (0, 0)),
        ],
        out_specs=[
            pl.BlockSpec((1, N, TW), lambda b: (b, 0, 0)),
            pl.BlockSpec((1, N, K), lambda b: (b, 0, 0)),
        ],
        out_shape=[
            jax.ShapeDtypeStruct((B, N, TW), jnp.float32),
            jax.ShapeDtypeStruct((B, N, K), jnp.int32),
        ],
    )(xp, posp, post, wqkv_p, wp1_p)


_NC, _NS = 2, 16
_NW = _NC * _NS
_RPW = R // _NW
_CH = 128
_NCHUNK = _RPW // _CH


@functools.cache
def _make_stage_b():
    mesh = plsc.VectorSubcoreMesh(core_axis_name="c", subcore_axis_name="s",
                                  num_cores=_NC, num_subcores=_NS)

    @functools.partial(
        pl.kernel,
        out_type=jax.ShapeDtypeStruct((R, TW), jnp.float32),
        mesh=mesh,
        scratch_types=[
            pltpu.VMEM((_RPW,), jnp.int32),
            pltpu.VMEM((_CH, TW), jnp.float32),
            pltpu.VMEM((_CH, TW), jnp.float32),
            pltpu.SemaphoreType.DMA,
            pltpu.SemaphoreType.DMA,
            pltpu.SemaphoreType.DMA,
            pltpu.SemaphoreType.DMA,
        ],
    )
    def _stage_b(tab_hbm, idx_hbm, out_hbm, idx_v, row_v0, row_v1,
                 gsem0, gsem1, ssem0, ssem1):
        wid = lax.axis_index("s") * _NC + lax.axis_index("c")
        base0 = wid * _RPW
        pltpu.sync_copy(idx_hbm.at[pl.ds(base0, _RPW)], idx_v)
        rows = (row_v0, row_v1)
        gsems = (gsem0, gsem1)
        ssems = (ssem0, ssem1)
        gd = [None] * _NCHUNK
        sd = [None] * _NCHUNK
        for c in range(min(2, _NCHUNK)):
            gd[c] = pltpu.async_copy(
                tab_hbm.at[idx_v.at[pl.ds(c * _CH, _CH)]],
                rows[c % 2], gsems[c % 2])
        for c in range(_NCHUNK):
            gd[c].wait()
            sd[c] = pltpu.async_copy(
                rows[c % 2], out_hbm.at[pl.ds(base0 + c * _CH, _CH)],
                ssems[c % 2])
            if c + 2 < _NCHUNK:
                sd[c].wait()
                gd[c + 2] = pltpu.async_copy(
                    tab_hbm.at[idx_v.at[pl.ds((c + 2) * _CH, _CH)]],
                    rows[c % 2], gsems[c % 2])
        for c in range(max(_NCHUNK - 2, 0), _NCHUNK):
            sd[c].wait()

    return _stage_b


_QB = 512


def _stage_c_body(tq_ref, gath_ref, wp2_ref, bp2_ref, wa1_ref, ba1_ref,
                  wa2_ref, ba2_ref, bp1_ref, out_ref):
    tq = tq_ref[...]
    q = tq[:, :D]
    a1q = tq[:, 3 * D:]
    gf = gath_ref[...].reshape(K * _QB, TW)
    kk = gf[:, D:2 * D]
    vv0 = gf[:, 2 * D:3 * D]
    a1j = gf[:, 3 * D:]

    qb = jnp.broadcast_to(q[None], (K, _QB, D)).reshape(K * _QB, D)
    a1qb = jnp.broadcast_to(a1q[None], (K, _QB, D)).reshape(K * _QB, D)

    pre = jnp.maximum(a1qb - a1j + bp1_ref[...], 0.0)
    pe = jnp.dot(pre, wp2_ref[...],
                 preferred_element_type=jnp.float32, precision=_MLP) \
        + bp2_ref[...]
    h = jnp.maximum(
        jnp.dot(qb - kk + pe, wa1_ref[...],
                preferred_element_type=jnp.float32, precision=_MLP)
        + ba1_ref[...], 0.0)
    sim = jnp.dot(h, wa2_ref[...],
                  preferred_element_type=jnp.float32, precision=_MLP) \
        + ba2_ref[...]
    vv = vv0 + pe

    sim3 = sim.reshape(K, _QB, D)
    vv3 = vv.reshape(K, _QB, D)
    m = sim3[0]
    num = vv3[0]
    den = jnp.ones_like(m)
    for k in range(1, K):
        s = sim3[k]
        m2 = jnp.maximum(m, s)
        a = jnp.exp(m - m2)
        e = jnp.exp(s - m2)
        num = num * a + e * vv3[k]
        den = den * a + e
        m = m2
    out_ref[...] = num / den


def _stage_c(tab_flat, gath, wp2, bp2, wa1, ba1, wa2, ba2, bp1):
    grid = (B * N) // _QB
    return pl.pallas_call(
        _stage_c_body,
        grid=(grid,),
        in_specs=[
            pl.BlockSpec((_QB, TW), lambda i: (i, 0)),
            pl.BlockSpec((K, _QB, TW), lambda i: (0, i, 0)),
            pl.BlockSpec((PH, D), lambda i: (0, 0)),
            pl.BlockSpec((1, D), lambda i: (0, 0)),
            pl.BlockSpec((D, D * H), lambda i: (0, 0)),
            pl.BlockSpec((1, D * H), lambda i: (0, 0)),
            pl.BlockSpec((D * H, D), lambda i: (0, 0)),
            pl.BlockSpec((1, D), lambda i: (0, 0)),
            pl.BlockSpec((1, PH), lambda i: (0, 0)),
        ],
        out_specs=pl.BlockSpec((_QB, D), lambda i: (i, 0)),
        out_shape=jax.ShapeDtypeStruct((B * N, D), jnp.float32),
    )(tab_flat, gath, wp2, bp2, wa1, ba1, wa2, ba2, bp1)


def kernel(x, pos, Wqkv, Wp1, bp1, Wp2, bp2, Wa1, ba1, Wa2, ba2):
    xp = jnp.pad(x, ((0, 0), (0, 0), (0, PPAD - D_IN)))
    posp = jnp.pad(pos, ((0, 0), (0, 0), (0, PPAD - D_IN)))
    post = jnp.transpose(posp, (0, 2, 1))
    wqkv_p = jnp.pad(Wqkv, ((0, PPAD - D_IN), (0, 0)))
    wp1_p = jnp.pad(Wp1, ((0, PPAD - D_IN), (0, 0)))

    tab, idx = _stage_a(xp, posp, post, wqkv_p, wp1_p)

    idx_t = jnp.transpose(idx, (2, 0, 1)).reshape(R)
    tab_flat = tab.reshape(B * N, TW)
    gath = _make_stage_b()(tab_flat, idx_t)

    agg = _stage_c(
        tab_flat, gath.reshape(K, B * N, TW),
        Wp2, bp2.reshape(1, D), Wa1, ba1.reshape(1, D * H),
        Wa2, ba2.reshape(1, D), bp1.reshape(1, PH))
    return agg.reshape(B, N, D)


# speedup vs baseline: 8.9735x; 1.0017x over previous
import functools

import jax
import jax.numpy as jnp
from jax import lax
from jax.experimental import pallas as pl
from jax.experimental.pallas import tpu as pltpu
from jax.experimental.pallas import tpu_sc as plsc

B, N, D_IN, D, H, K, PH = 4, 512, 3, 64, 4, 16, 64
PPAD = 16
TW = 4 * D
R = B * N * K
_HI = jax.lax.Precision.HIGHEST
_MLP = jax.lax.Precision.DEFAULT


def _stage_a_body(xp_ref, posp_ref, post_ref, wqkv_ref, wp1_ref,
                  tab_ref, idx_ref):
    b = pl.program_id(0)
    xp = xp_ref[0]
    P = posp_ref[0]
    PT = post_ref[0]
    qkv = jnp.dot(xp, wqkv_ref[...],
                  preferred_element_type=jnp.float32, precision=_HI)
    tab_ref[0, :, :3 * D] = qkv
    tab_ref[0, :, 3 * D:] = jnp.dot(
        P, wp1_ref[...], preferred_element_type=jnp.float32, precision=_HI)

    t0 = P[:, 0:1] - PT[0:1, :]
    t1 = P[:, 1:2] - PT[1:2, :]
    t2 = P[:, 2:3] - PT[2:3, :]
    nd = jnp.sqrt(t0 * t0 + t1 * t1 + t2 * t2)

    jj = lax.broadcasted_iota(jnp.int32, (N, N), 1).astype(jnp.float32)

    off = jnp.int32(N) * b
    for t in range(K):
        m = jnp.min(nd, axis=1, keepdims=True)
        cand = jnp.where(nd <= m, jj, jnp.float32(N))
        amin = jnp.min(cand, axis=1, keepdims=True)
        idx_ref[0, :, pl.ds(t, 1)] = amin.astype(jnp.int32) + off
        nd = jnp.where(jj == amin, jnp.float32(3e38), nd)


def _stage_a(xp, posp, post, wqkv_p, wp1_p):
    return pl.pallas_call(
        _stage_a_body,
        grid=(B,),
        in_specs=[
            pl.BlockSpec((1, N, PPAD), lambda b: (b, 0, 0)),
            pl.BlockSpec((1, N, PPAD), lambda b: (b, 0, 0)),
            pl.BlockSpec((1, PPAD, N), lambda b: (b, 0, 0)),
            pl.BlockSpec((PPAD, 3 * D), lambda b: (0, 0)),
            pl.BlockSpec((PPAD, PH), lambda b: (0, 0)),
        ],
        out_specs=[
            pl.BlockSpec((1, N, TW), lambda b: (b, 0, 0)),
            pl.BlockSpec((1, N, K), lambda b: (b, 0, 0)),
        ],
        out_shape=[
            jax.ShapeDtypeStruct((B, N, TW), jnp.float32),
            jax.ShapeDtypeStruct((B, N, K), jnp.int32),
        ],
    )(xp, posp, post, wqkv_p, wp1_p)


_NC, _NS = 2, 16
_NW = _NC * _NS
_RPW = R // _NW
_CH = 128
_NCHUNK = _RPW // _CH


@functools.cache
def _make_stage_b():
    mesh = plsc.VectorSubcoreMesh(core_axis_name="c", subcore_axis_name="s",
                                  num_cores=_NC, num_subcores=_NS)
    _NB = 3

    @functools.partial(
        pl.kernel,
        out_type=jax.ShapeDtypeStruct((R, TW), jnp.float32),
        mesh=mesh,
        scratch_types=[
            pltpu.VMEM((_RPW,), jnp.int32),
            pltpu.VMEM((_CH, TW), jnp.float32),
            pltpu.VMEM((_CH, TW), jnp.float32),
            pltpu.VMEM((_CH, TW), jnp.float32),
            pltpu.SemaphoreType.DMA,
            pltpu.SemaphoreType.DMA,
            pltpu.SemaphoreType.DMA,
            pltpu.SemaphoreType.DMA,
            pltpu.SemaphoreType.DMA,
            pltpu.SemaphoreType.DMA,
        ],
    )
    def _stage_b(tab_hbm, idx_hbm, out_hbm, idx_v, row_v0, row_v1,
                 row_v2, gsem0, gsem1, gsem2, ssem0, ssem1, ssem2):
        wid = lax.axis_index("s") * _NC + lax.axis_index("c")
        base0 = wid * _RPW
        pltpu.sync_copy(idx_hbm.at[pl.ds(base0, _RPW)], idx_v)
        rows = (row_v0, row_v1, row_v2)
        gsems = (gsem0, gsem1, gsem2)
        ssems = (ssem0, ssem1, ssem2)
        gd = [None] * _NCHUNK
        sd = [None] * _NCHUNK
        for c in range(min(_NB, _NCHUNK)):
            gd[c] = pltpu.async_copy(
                tab_hbm.at[idx_v.at[pl.ds(c * _CH, _CH)]],
                rows[c % _NB], gsems[c % _NB])
        for c in range(_NCHUNK):
            gd[c].wait()
            sd[c] = pltpu.async_copy(
                rows[c % _NB], out_hbm.at[pl.ds(base0 + c * _CH, _CH)],
                ssems[c % _NB])
            if c + _NB < _NCHUNK:
                sd[c].wait()
                gd[c + _NB] = pltpu.async_copy(
                    tab_hbm.at[idx_v.at[pl.ds((c + _NB) * _CH, _CH)]],
                    rows[c % _NB], gsems[c % _NB])
        for c in range(max(_NCHUNK - _NB, 0), _NCHUNK):
            sd[c].wait()

    return _stage_b


_QB = 512


def _stage_c_body(tq_ref, gath_ref, wp2_ref, bp2_ref, wa1_ref, ba1_ref,
                  wa2_ref, ba2_ref, bp1_ref, out_ref):
    tq = tq_ref[...]
    q = tq[:, :D]
    a1q = tq[:, 3 * D:]
    gf = gath_ref[...].reshape(K * _QB, TW)
    kk = gf[:, D:2 * D]
    vv0 = gf[:, 2 * D:3 * D]
    a1j = gf[:, 3 * D:]

    qb = jnp.broadcast_to(q[None], (K, _QB, D)).reshape(K * _QB, D)
    a1qb = jnp.broadcast_to(a1q[None], (K, _QB, D)).reshape(K * _QB, D)

    pre = jnp.maximum(a1qb - a1j + bp1_ref[...], 0.0)
    pe = jnp.dot(pre, wp2_ref[...],
                 preferred_element_type=jnp.float32, precision=_MLP) \
        + bp2_ref[...]
    h = jnp.maximum(
        jnp.dot(qb - kk + pe, wa1_ref[...],
                preferred_element_type=jnp.float32, precision=_MLP)
        + ba1_ref[...], 0.0)
    sim = jnp.dot(h, wa2_ref[...],
                  preferred_element_type=jnp.float32, precision=_MLP) \
        + ba2_ref[...]
    vv = vv0 + pe

    sim3 = sim.reshape(K, _QB, D)
    vv3 = vv.reshape(K, _QB, D)
    m = sim3[0]
    num = vv3[0]
    den = jnp.ones_like(m)
    for k in range(1, K):
        s = sim3[k]
        m2 = jnp.maximum(m, s)
        a = jnp.exp(m - m2)
        e = jnp.exp(s - m2)
        num = num * a + e * vv3[k]
        den = den * a + e
        m = m2
    out_ref[...] = num / den


def _stage_c(tab_flat, gath, wp2, bp2, wa1, ba1, wa2, ba2, bp1):
    grid = (B * N) // _QB
    return pl.pallas_call(
        _stage_c_body,
        grid=(grid,),
        in_specs=[
            pl.BlockSpec((_QB, TW), lambda i: (i, 0)),
            pl.BlockSpec((K, _QB, TW), lambda i: (0, i, 0)),
            pl.BlockSpec((PH, D), lambda i: (0, 0)),
            pl.BlockSpec((1, D), lambda i: (0, 0)),
            pl.BlockSpec((D, D * H), lambda i: (0, 0)),
            pl.BlockSpec((1, D * H), lambda i: (0, 0)),
            pl.BlockSpec((D * H, D), lambda i: (0, 0)),
            pl.BlockSpec((1, D), lambda i: (0, 0)),
            pl.BlockSpec((1, PH), lambda i: (0, 0)),
        ],
        out_specs=pl.BlockSpec((_QB, D), lambda i: (i, 0)),
        out_shape=jax.ShapeDtypeStruct((B * N, D), jnp.float32),
    )(tab_flat, gath, wp2, bp2, wa1, ba1, wa2, ba2, bp1)


def kernel(x, pos, Wqkv, Wp1, bp1, Wp2, bp2, Wa1, ba1, Wa2, ba2):
    xp = jnp.pad(x, ((0, 0), (0, 0), (0, PPAD - D_IN)))
    posp = jnp.pad(pos, ((0, 0), (0, 0), (0, PPAD - D_IN)))
    post = jnp.transpose(posp, (0, 2, 1))
    wqkv_p = jnp.pad(Wqkv, ((0, PPAD - D_IN), (0, 0)))
    wp1_p = jnp.pad(Wp1, ((0, PPAD - D_IN), (0, 0)))

    tab, idx = _stage_a(xp, posp, post, wqkv_p, wp1_p)

    idx_t = jnp.transpose(idx, (2, 0, 1)).reshape(R)
    tab_flat = tab.reshape(B * N, TW)
    gath = _make_stage_b()(tab_flat, idx_t)

    agg = _stage_c(
        tab_flat, gath.reshape(K, B * N, TW),
        Wp2, bp2.reshape(1, D), Wa1, ba1.reshape(1, D * H),
        Wa2, ba2.reshape(1, D), bp1.reshape(1, PH))
    return agg.reshape(B, N, D)


# speedup vs baseline: 9.7116x; 1.0822x over previous
import functools

import jax
import jax.numpy as jnp
from jax import lax
from jax.experimental import pallas as pl
from jax.experimental.pallas import tpu as pltpu
from jax.experimental.pallas import tpu_sc as plsc

B, N, D_IN, D, H, K, PH = 4, 512, 3, 64, 4, 16, 64
PPAD = 16
TW = 4 * D
GROUPS = 2
NB_G = B // GROUPS
R_G = NB_G * N * K
_HI = jax.lax.Precision.HIGHEST
_MLP = jax.lax.Precision.DEFAULT

_NC, _NS = 2, 16
_NW = _NC * _NS
_RPW = R_G // _NW
_CH = 128
_NCHUNK = _RPW // _CH
_QB = NB_G * N


def _stage_a_body(xp_ref, posp_ref, post_ref, wqkv_ref, wp1_ref,
                  tab_ref, idx_ref):
    b = pl.program_id(0)
    xp = xp_ref[0]
    P = posp_ref[0]
    PT = post_ref[0]
    qkv = jnp.dot(xp, wqkv_ref[...],
                  preferred_element_type=jnp.float32, precision=_HI)
    tab_ref[0, :, :3 * D] = qkv
    tab_ref[0, :, 3 * D:] = jnp.dot(
        P, wp1_ref[...], preferred_element_type=jnp.float32, precision=_HI)

    t0 = P[:, 0:1] - PT[0:1, :]
    t1 = P[:, 1:2] - PT[1:2, :]
    t2 = P[:, 2:3] - PT[2:3, :]
    nd = jnp.sqrt(t0 * t0 + t1 * t1 + t2 * t2)

    jj = lax.broadcasted_iota(jnp.int32, (N, N), 1).astype(jnp.float32)

    off = jnp.int32(N) * b
    for t in range(K):
        m = jnp.min(nd, axis=1, keepdims=True)
        cand = jnp.where(nd <= m, jj, jnp.float32(N))
        amin = jnp.min(cand, axis=1, keepdims=True)
        idx_ref[0, :, pl.ds(t, 1)] = amin.astype(jnp.int32) + off
        nd = jnp.where(jj == amin, jnp.float32(3e38), nd)


def _stage_a(xp, posp, post, wqkv_p, wp1_p):
    return pl.pallas_call(
        _stage_a_body,
        grid=(NB_G,),
        in_specs=[
            pl.BlockSpec((1, N, PPAD), lambda b: (b, 0, 0)),
            pl.BlockSpec((1, N, PPAD), lambda b: (b, 0, 0)),
            pl.BlockSpec((1, PPAD, N), lambda b: (b, 0, 0)),
            pl.BlockSpec((PPAD, 3 * D), lambda b: (0, 0)),
            pl.BlockSpec((PPAD, PH), lambda b: (0, 0)),
        ],
        out_specs=[
            pl.BlockSpec((1, N, TW), lambda b: (b, 0, 0)),
            pl.BlockSpec((1, N, K), lambda b: (b, 0, 0)),
        ],
        out_shape=[
            jax.ShapeDtypeStruct((NB_G, N, TW), jnp.float32),
            jax.ShapeDtypeStruct((NB_G, N, K), jnp.int32),
        ],
    )(xp, posp, post, wqkv_p, wp1_p)


@functools.cache
def _make_stage_b():
    mesh = plsc.VectorSubcoreMesh(core_axis_name="c", subcore_axis_name="s",
                                  num_cores=_NC, num_subcores=_NS)
    _NB = 3

    @functools.partial(
        pl.kernel,
        out_type=jax.ShapeDtypeStruct((R_G, TW), jnp.float32),
        mesh=mesh,
        scratch_types=[
            pltpu.VMEM((_RPW,), jnp.int32),
            pltpu.VMEM((_CH, TW), jnp.float32),
            pltpu.VMEM((_CH, TW), jnp.float32),
            pltpu.VMEM((_CH, TW), jnp.float32),
            pltpu.SemaphoreType.DMA,
            pltpu.SemaphoreType.DMA,
            pltpu.SemaphoreType.DMA,
            pltpu.SemaphoreType.DMA,
            pltpu.SemaphoreType.DMA,
            pltpu.SemaphoreType.DMA,
        ],
    )
    def _stage_b(tab_hbm, idx_hbm, out_hbm, idx_v, row_v0, row_v1,
                 row_v2, gsem0, gsem1, gsem2, ssem0, ssem1, ssem2):
        wid = lax.axis_index("s") * _NC + lax.axis_index("c")
        base0 = wid * _RPW
        pltpu.sync_copy(idx_hbm.at[pl.ds(base0, _RPW)], idx_v)
        rows = (row_v0, row_v1, row_v2)
        gsems = (gsem0, gsem1, gsem2)
        ssems = (ssem0, ssem1, ssem2)
        gd = [None] * _NCHUNK
        sd = [None] * _NCHUNK
        for c in range(min(_NB, _NCHUNK)):
            gd[c] = pltpu.async_copy(
                tab_hbm.at[idx_v.at[pl.ds(c * _CH, _CH)]],
                rows[c % _NB], gsems[c % _NB])
        for c in range(_NCHUNK):
            gd[c].wait()
            sd[c] = pltpu.async_copy(
                rows[c % _NB], out_hbm.at[pl.ds(base0 + c * _CH, _CH)],
                ssems[c % _NB])
            if c + _NB < _NCHUNK:
                sd[c].wait()
                gd[c + _NB] = pltpu.async_copy(
                    tab_hbm.at[idx_v.at[pl.ds((c + _NB) * _CH, _CH)]],
                    rows[c % _NB], gsems[c % _NB])
        for c in range(max(_NCHUNK - _NB, 0), _NCHUNK):
            sd[c].wait()

    return _stage_b


def _stage_c_body(tq_ref, gath_ref, wp2_ref, bp2_ref, wa1_ref, ba1_ref,
                  wa2_ref, ba2_ref, bp1_ref, out_ref):
    tq = tq_ref[...]
    q = tq[:, :D]
    a1q = tq[:, 3 * D:]
    gf = gath_ref[...].reshape(K * _QB, TW)
    kk = gf[:, D:2 * D]
    vv0 = gf[:, 2 * D:3 * D]
    a1j = gf[:, 3 * D:]

    qb = jnp.broadcast_to(q[None], (K, _QB, D)).reshape(K * _QB, D)
    a1qb = jnp.broadcast_to(a1q[None], (K, _QB, D)).reshape(K * _QB, D)

    pre = jnp.maximum(a1qb - a1j + bp1_ref[...], 0.0)
    pe = jnp.dot(pre, wp2_ref[...],
                 preferred_element_type=jnp.float32, precision=_MLP) \
        + bp2_ref[...]
    h = jnp.maximum(
        jnp.dot(qb - kk + pe, wa1_ref[...],
                preferred_element_type=jnp.float32, precision=_MLP)
        + ba1_ref[...], 0.0)
    sim = jnp.dot(h, wa2_ref[...],
                  preferred_element_type=jnp.float32, precision=_MLP) \
        + ba2_ref[...]
    vv = vv0 + pe

    sim3 = sim.reshape(K, _QB, D)
    vv3 = vv.reshape(K, _QB, D)
    m = sim3[0]
    num = vv3[0]
    den = jnp.ones_like(m)
    for k in range(1, K):
        s = sim3[k]
        m2 = jnp.maximum(m, s)
        a = jnp.exp(m - m2)
        e = jnp.exp(s - m2)
        num = num * a + e * vv3[k]
        den = den * a + e
        m = m2
    out_ref[...] = num / den


def _stage_c(tab_flat, gath, wp2, bp2, wa1, ba1, wa2, ba2, bp1):
    return pl.pallas_call(
        _stage_c_body,
        grid=(1,),
        in_specs=[
            pl.BlockSpec((_QB, TW), lambda i: (i, 0)),
            pl.BlockSpec((K, _QB, TW), lambda i: (0, i, 0)),
            pl.BlockSpec((PH, D), lambda i: (0, 0)),
            pl.BlockSpec((1, D), lambda i: (0, 0)),
            pl.BlockSpec((D, D * H), lambda i: (0, 0)),
            pl.BlockSpec((1, D * H), lambda i: (0, 0)),
            pl.BlockSpec((D * H, D), lambda i: (0, 0)),
            pl.BlockSpec((1, D), lambda i: (0, 0)),
            pl.BlockSpec((1, PH), lambda i: (0, 0)),
        ],
        out_specs=pl.BlockSpec((_QB, D), lambda i: (i, 0)),
        out_shape=jax.ShapeDtypeStruct((_QB, D), jnp.float32),
    )(tab_flat, gath, wp2, bp2, wa1, ba1, wa2, ba2, bp1)


def kernel(x, pos, Wqkv, Wp1, bp1, Wp2, bp2, Wa1, ba1, Wa2, ba2):
    xp = jnp.pad(x, ((0, 0), (0, 0), (0, PPAD - D_IN)))
    posp = jnp.pad(pos, ((0, 0), (0, 0), (0, PPAD - D_IN)))
    post = jnp.transpose(posp, (0, 2, 1))
    wqkv_p = jnp.pad(Wqkv, ((0, PPAD - D_IN), (0, 0)))
    wp1_p = jnp.pad(Wp1, ((0, PPAD - D_IN), (0, 0)))
    bp1r = bp1.reshape(1, PH)
    bp2r = bp2.reshape(1, D)
    ba1r = ba1.reshape(1, D * H)
    ba2r = ba2.reshape(1, D)

    stage_b = _make_stage_b()
    outs = []
    for g in range(GROUPS):
        lo, hi = g * NB_G, (g + 1) * NB_G
        tab, idx = _stage_a(xp[lo:hi], posp[lo:hi], post[lo:hi],
                            wqkv_p, wp1_p)
        idx_t = jnp.transpose(idx, (2, 0, 1)).reshape(R_G)
        tab_flat = tab.reshape(NB_G * N, TW)
        gath = stage_b(tab_flat, idx_t)
        outs.append(_stage_c(
            tab_flat, gath.reshape(K, NB_G * N, TW),
            Wp2, bp2r, Wa1, ba1r, Wa2, ba2r, bp1r))
    return jnp.concatenate(outs, axis=0).reshape(B, N, D)
